# Initial kernel scaffold; baseline (speedup 1.0000x reference)
import jax
import jax.numpy as jnp
from jax.experimental import pallas as pl


def kernel(x, edge_index, W1, b1, W2, b2):
    raise NotImplementedError("write your pallas kernel here")


# speedup vs baseline: 105.6451x; 105.6451x over previous
import functools

import jax
import jax.numpy as jnp
from jax import lax
from jax.experimental import pallas as pl
from jax.experimental.pallas import tpu as pltpu
from jax.experimental.pallas import tpu_sc as plsc

N_NODES = 100000
NC, NS = 2, 16
NW = NC * NS
NB = 128
KS = 16
NTW = 3136
NP = NTW * NW
NPC = NP // NS

_MESH = plsc.VectorSubcoreMesh(
    core_axis_name="c", subcore_axis_name="s", num_cores=NC, num_subcores=NS)
_PARAMS = pltpu.CompilerParams(use_tc_tiling_on_sc=False,
                               needs_layout_passes=False)

_F32 = jnp.float32
_I32 = jnp.int32


def _wid():
    return lax.axis_index("c") * NS + lax.axis_index("s")


def _iota16():
    return lax.broadcasted_iota(_I32, (16,), 0)


def _full16(v):
    return jnp.full((16,), v, dtype=_I32)


def _rsqrt16(v):
    i = plsc.bitcast(v, _I32)
    i = jnp.int32(0x5F3759DF) - lax.shift_right_arithmetic(i, 1)
    y = plsc.bitcast(i, _F32)
    for _ in range(3):
        y = y * (1.5 - 0.5 * v * y * y)
    return y


def _bf16r(v):
    u = plsc.bitcast(v, _I32)
    r = u + jnp.int32(0x7FFF) + (lax.shift_right_logical(u, 16) & 1)
    r = r & jnp.int32(-65536)
    return plsc.bitcast(r, _F32)


def _bcast(ref, *idx):
    return plsc.load_gather(ref, [_full16(i) for i in idx])


def _make_deg(n_iters):
    @functools.partial(
        pl.kernel,
        out_type=jax.ShapeDtypeStruct((NC, NP), _F32),
        mesh=_MESH,
        compiler_params=_PARAMS,
        scratch_types=[
            pltpu.VMEM((KS, NB), _I32),
            pltpu.VMEM((NP,), _F32),
            pltpu.VMEM((NP // 128, ), _F32),
            pltpu.VMEM((NP // 128, ), _F32),
            pltpu.VMEM_SHARED((NS, NP // 8), _F32),
        ],
    )
    def deg_kernel(dst2d, zeros1, degp, idxb, hist, abuf, bbuf, spm):
        c = lax.axis_index("c")
        s = lax.axis_index("s")
        w = _wid()
        ones16 = jnp.full((16,), 1.0, dtype=_F32)

        pltpu.sync_copy(zeros1, hist)
        rows_per_worker = n_iters * KS

        def body(g, _):
            rbase = w * rows_per_worker + g * KS
            pltpu.sync_copy(dst2d.at[pl.ds(rbase, KS)], idxb)
            for j in range(KS):
                def inner(k, _):
                    iv = idxb[j, pl.ds(k * 16, 16)]
                    plsc.addupdate_scatter(hist, [iv], ones16)
                    return 0
                lax.fori_loop(0, NB // 16, inner, 0)
            return 0
        lax.fori_loop(0, n_iters, body, 0)

        ch = NP // 8
        nred = ch // NS
        for r in range(8):
            pltpu.sync_copy(hist.at[pl.ds(r * ch, ch)], spm.at[s])
            plsc.subcore_barrier()
            pltpu.sync_copy(spm.at[0, pl.ds(s * nred, nred)], abuf)
            for slot in range(1, NS):
                pltpu.sync_copy(spm.at[slot, pl.ds(s * nred, nred)], bbuf)

                def acc(i, _):
                    sl = pl.ds(i * 16, 16)
                    abuf[sl] = abuf[sl] + bbuf[sl]
                    return 0
                lax.fori_loop(0, nred // 16, acc, 0)
            pltpu.sync_copy(abuf, degp.at[c, pl.ds(r * ch + s * nred, nred)])
            plsc.subcore_barrier()

    return deg_kernel


@functools.partial(
    pl.kernel,
    out_type=(jax.ShapeDtypeStruct((NP, 8), _F32),
              jax.ShapeDtypeStruct((NP,), _F32)),
    mesh=_MESH,
    compiler_params=_PARAMS,
    scratch_types=[
        pltpu.VMEM((NTW,), _F32),
        pltpu.VMEM((NTW,), _F32),
        pltpu.VMEM((5, NTW), _F32),
        pltpu.VMEM((NTW,), _F32),
        pltpu.VMEM((NTW, 8), _F32),
    ],
)
def _prep_kernel(degp, xt, zeros8, t1, dis, d0b, d1b, colsb, disb, t1b):
    base = _wid() * NTW
    pltpu.sync_copy(degp.at[0, pl.ds(base, NTW)], d0b)
    pltpu.sync_copy(degp.at[1, pl.ds(base, NTW)], d1b)
    for k in range(5):
        pltpu.sync_copy(xt.at[k, pl.ds(base, NTW)], colsb.at[k])
    pltpu.sync_copy(zeros8.at[pl.ds(base, NTW), :], t1b)
    io0 = _iota16()

    def body(i, _):
        sl = pl.ds(i * 16, 16)
        dv = d0b[sl] + d1b[sl] + 1.0
        y = _rsqrt16(dv)
        disb[sl] = y
        rowi = io0 + i * 16
        for k in range(5):
            plsc.store_scatter(t1b, [rowi, _full16(k)], _bf16r(colsb[k, sl]) * y)
        return 0
    lax.fori_loop(0, NTW // 16, body, 0)

    pltpu.sync_copy(disb, dis.at[pl.ds(base, NTW)])
    pltpu.sync_copy(t1b, t1.at[pl.ds(base, NTW), :])


def _make_agg(n_iters):
    @functools.partial(
        pl.kernel,
        out_type=jax.ShapeDtypeStruct((NC * NP, 8), _F32),
        mesh=_MESH,
        compiler_params=_PARAMS,
        scratch_types=[
            pltpu.VMEM((KS, NB), _I32),
            pltpu.VMEM((KS, NB), _I32),
            pltpu.VMEM((KS, NB, 8), _F32),
            pltpu.VMEM_SHARED((NP, 8), _F32),
            pltpu.SemaphoreType.DMA,
            pltpu.SemaphoreType.DMA,
        ],
    )
    def agg_kernel(table, src2d, dst2d, zeros8, part,
                   sidx, didx, rows, acc, semg, sems):
        c = lax.axis_index("c")
        s = lax.axis_index("s")
        w = _wid()

        pltpu.sync_copy(zeros8.at[pl.ds(s * NPC, NPC), :],
                        acc.at[pl.ds(s * NPC, NPC), :])
        plsc.subcore_barrier()

        rows_per_worker = n_iters * KS

        def body(g, _):
            rbase = w * rows_per_worker + g * KS
            pltpu.sync_copy(src2d.at[pl.ds(rbase, KS)], sidx)
            pltpu.sync_copy(dst2d.at[pl.ds(rbase, KS)], didx)
            gd = [
                pltpu.async_copy(table.at[sidx.at[j]], rows.at[j], semg)
                for j in range(KS)
            ]
            for d in gd:
                d.wait()
            sd = [
                pltpu.async_copy(rows.at[j], acc.at[didx.at[j]], sems, add=True)
                for j in range(KS)
            ]
            for d in sd:
                d.wait()
            return 0
        lax.fori_loop(0, n_iters, body, 0)

        plsc.subcore_barrier()
        pltpu.sync_copy(acc.at[pl.ds(s * NPC, NPC), :],
                        part.at[pl.ds(c * NP + s * NPC, NPC), :])

    return agg_kernel


@functools.partial(
    pl.kernel,
    out_type=jax.ShapeDtypeStruct((NP, 8), _F32),
    mesh=_MESH,
    compiler_params=_PARAMS,
    scratch_types=[
        pltpu.VMEM((NTW, 8), _F32),
        pltpu.VMEM((NTW, 8), _F32),
        pltpu.VMEM((NTW, 8), _F32),
        pltpu.VMEM((NTW,), _F32),
        pltpu.VMEM((NTW, 8), _F32),
        pltpu.VMEM((80,), _F32),
        pltpu.VMEM((16,), _F32),
        pltpu.VMEM((48,), _F32),
    ],
)
def _dense1_kernel(p1p, t1, dis, w1, b1, w2, zeros8, t2,
                   p0b, p1b, t1b, disb, t2b, w1b, b1b, w2b):
    base = _wid() * NTW
    pltpu.sync_copy(p1p.at[pl.ds(base, NTW), :], p0b)
    pltpu.sync_copy(p1p.at[pl.ds(NP + base, NTW), :], p1b)
    pltpu.sync_copy(t1.at[pl.ds(base, NTW), :], t1b)
    pltpu.sync_copy(dis.at[pl.ds(base, NTW)], disb)
    pltpu.sync_copy(w1, w1b)
    pltpu.sync_copy(b1, b1b)
    pltpu.sync_copy(w2, w2b)
    pltpu.sync_copy(zeros8.at[pl.ds(base, NTW), :], t2b)
    for t in range(5):
        w1b[pl.ds(t * 16, 16)] = _bf16r(w1b[pl.ds(t * 16, 16)])
    for t in range(3):
        w2b[pl.ds(t * 16, 16)] = _bf16r(w2b[pl.ds(t * 16, 16)])
    io0 = _iota16()
    zero16 = jnp.zeros((16,), dtype=_F32)

    def body(i, _):
        sl = pl.ds(i * 16, 16)
        y = disb[sl]
        rowi = io0 + i * 16
        u = []
        for cidx in range(5):
            fc = _full16(cidx)
            col = (plsc.load_gather(p0b, [rowi, fc])
                   + plsc.load_gather(p1b, [rowi, fc])
                   + plsc.load_gather(t1b, [rowi, fc]))
            u.append(y * col)

        def col_body(j, accs):
            a0, a1, a2 = accs
            jv = jnp.full((16,), j, dtype=_I32)
            o = plsc.load_gather(b1b, [jv])
            for cidx in range(5):
                o = o + u[cidx] * plsc.load_gather(w1b, [jv + cidx * 16])
            r = _bf16r(jnp.maximum(o, 0.0))
            j3 = j * 3
            a0 = a0 + r * plsc.load_gather(w2b, [jnp.full((16,), j3, _I32)])
            a1 = a1 + r * plsc.load_gather(w2b, [jnp.full((16,), j3 + 1, _I32)])
            a2 = a2 + r * plsc.load_gather(w2b, [jnp.full((16,), j3 + 2, _I32)])
            return (a0, a1, a2)
        a0, a1, a2 = lax.fori_loop(0, 16, col_body, (zero16, zero16, zero16))
        plsc.store_scatter(t2b, [rowi, _full16(0)], a0 * y)
        plsc.store_scatter(t2b, [rowi, _full16(1)], a1 * y)
        plsc.store_scatter(t2b, [rowi, _full16(2)], a2 * y)
        return 0
    lax.fori_loop(0, NTW // 16, body, 0)

    pltpu.sync_copy(t2b, t2.at[pl.ds(base, NTW), :])


@functools.partial(
    pl.kernel,
    out_type=jax.ShapeDtypeStruct((3, NP), _F32),
    mesh=_MESH,
    compiler_params=_PARAMS,
    scratch_types=[
        pltpu.VMEM((NTW, 8), _F32),
        pltpu.VMEM((NTW, 8), _F32),
        pltpu.VMEM((NTW, 8), _F32),
        pltpu.VMEM((NTW,), _F32),
        pltpu.VMEM((4, NTW), _F32),
        pltpu.VMEM((16,), _F32),
    ],
)
def _dense2_kernel(p2p, t2, dis, b2, outc,
                   p0b, p1b, t2b, disb, ocb, b2b):
    base = _wid() * NTW
    pltpu.sync_copy(p2p.at[pl.ds(base, NTW), :], p0b)
    pltpu.sync_copy(p2p.at[pl.ds(NP + base, NTW), :], p1b)
    pltpu.sync_copy(t2.at[pl.ds(base, NTW), :], t2b)
    pltpu.sync_copy(dis.at[pl.ds(base, NTW)], disb)
    pltpu.sync_copy(b2, b2b)
    io0 = _iota16()

    def body(i, _):
        sl = pl.ds(i * 16, 16)
        y = disb[sl]
        rowi = io0 + i * 16
        for k in range(3):
            fc = _full16(k)
            col = (plsc.load_gather(p0b, [rowi, fc])
                   + plsc.load_gather(p1b, [rowi, fc])
                   + plsc.load_gather(t2b, [rowi, fc]))
            ocb[k, sl] = y * col + _bcast(b2b, k)
        return 0
    lax.fori_loop(0, NTW // 16, body, 0)

    for k in range(3):
        pltpu.sync_copy(ocb.at[k], outc.at[k, pl.ds(base, NTW)])


def kernel(x, edge_index, W1, b1, W2, b2):
    n_edges = edge_index.shape[1]
    epw = NB * KS
    n_iters = -(-n_edges // (NW * epw))
    ep = n_iters * NW * epw

    pad = jnp.full((2, ep - n_edges), N_NODES, dtype=edge_index.dtype)
    eip = jnp.concatenate([edge_index.astype(_I32), pad], axis=1)
    src2d = eip[0].reshape(ep // NB, NB)
    dst2d = eip[1].reshape(ep // NB, NB)

    xt = jnp.pad(x, ((0, NP - x.shape[0]), (0, 0))).T
    z1 = jnp.zeros((NP,), _F32)
    z8 = jnp.zeros((NP, 8), _F32)

    agg = _make_agg(n_iters)
    degp = _make_deg(n_iters)(dst2d, z1)
    t1, dis = _prep_kernel(degp, xt, z8)
    p1p = agg(t1, src2d, dst2d, z8)
    t2 = _dense1_kernel(p1p, t1, dis, W1.reshape(-1), b1, W2.reshape(-1), z8)
    p2p = agg(t2, src2d, dst2d, z8)
    outc = _dense2_kernel(p2p, t2, dis, jnp.pad(b2, (0, 13)))
    return outc[:, :N_NODES].T


# speedup vs baseline: 161.2368x; 1.5262x over previous
import functools

import jax
import jax.numpy as jnp
from jax import lax
from jax.experimental import pallas as pl
from jax.experimental.pallas import tpu as pltpu
from jax.experimental.pallas import tpu_sc as plsc

N_NODES = 100000
NC, NS = 2, 16
NW = NC * NS
NB = 128
KS = 16
NTW = 3136
NP = NTW * NW
NPC = NP // NS

_MESH = plsc.VectorSubcoreMesh(
    core_axis_name="c", subcore_axis_name="s", num_cores=NC, num_subcores=NS)
_PARAMS = pltpu.CompilerParams(use_tc_tiling_on_sc=False,
                               needs_layout_passes=False)

_F32 = jnp.float32
_I32 = jnp.int32


def _wid():
    return lax.axis_index("c") * NS + lax.axis_index("s")


def _iota16():
    return lax.broadcasted_iota(_I32, (16,), 0)


def _full16(v):
    return jnp.full((16,), v, dtype=_I32)


def _rsqrt16(v):
    i = plsc.bitcast(v, _I32)
    i = jnp.int32(0x5F3759DF) - lax.shift_right_arithmetic(i, 1)
    y = plsc.bitcast(i, _F32)
    for _ in range(3):
        y = y * (1.5 - 0.5 * v * y * y)
    return y


def _bf16r(v):
    u = plsc.bitcast(v, _I32)
    r = u + jnp.int32(0x7FFF) + (lax.shift_right_logical(u, 16) & 1)
    r = r & jnp.int32(-65536)
    return plsc.bitcast(r, _F32)


def _bcast(ref, *idx):
    return plsc.load_gather(ref, [_full16(i) for i in idx])


def _make_deg(n_iters):
    @functools.partial(
        pl.kernel,
        out_type=jax.ShapeDtypeStruct((NC, NP), _F32),
        mesh=_MESH,
        compiler_params=_PARAMS,
        scratch_types=[
            pltpu.VMEM((KS, NB), _I32),
            pltpu.VMEM((NP,), _F32),
            pltpu.VMEM((NP // 128, ), _F32),
            pltpu.VMEM((NP // 128, ), _F32),
            pltpu.VMEM_SHARED((NS, NP // 8), _F32),
        ],
    )
    def deg_kernel(dst2d, zeros1, degp, idxb, hist, abuf, bbuf, spm):
        c = lax.axis_index("c")
        s = lax.axis_index("s")
        w = _wid()
        ones16 = jnp.full((16,), 1.0, dtype=_F32)

        pltpu.sync_copy(zeros1, hist)
        rows_per_worker = n_iters * KS

        def body(g, _):
            rbase = w * rows_per_worker + g * KS
            pltpu.sync_copy(dst2d.at[pl.ds(rbase, KS)], idxb)
            for j in range(KS):
                def inner(k, _):
                    iv = idxb[j, pl.ds(k * 16, 16)]
                    plsc.addupdate_scatter(hist, [iv], ones16)
                    return 0
                lax.fori_loop(0, NB // 16, inner, 0)
            return 0
        lax.fori_loop(0, n_iters, body, 0)

        ch = NP // 8
        nred = ch // NS
        for r in range(8):
            pltpu.sync_copy(hist.at[pl.ds(r * ch, ch)], spm.at[s])
            plsc.subcore_barrier()
            pltpu.sync_copy(spm.at[0, pl.ds(s * nred, nred)], abuf)
            for slot in range(1, NS):
                pltpu.sync_copy(spm.at[slot, pl.ds(s * nred, nred)], bbuf)

                def acc(i, _):
                    sl = pl.ds(i * 16, 16)
                    abuf[sl] = abuf[sl] + bbuf[sl]
                    return 0
                lax.fori_loop(0, nred // 16, acc, 0)
            pltpu.sync_copy(abuf, degp.at[c, pl.ds(r * ch + s * nred, nred)])
            plsc.subcore_barrier()

    return deg_kernel


@functools.partial(
    pl.kernel,
    out_type=(jax.ShapeDtypeStruct((NP, 8), _F32),
              jax.ShapeDtypeStruct((NP,), _F32)),
    mesh=_MESH,
    compiler_params=_PARAMS,
    scratch_types=[
        pltpu.VMEM((NTW,), _F32),
        pltpu.VMEM((NTW,), _F32),
        pltpu.VMEM((5, NTW), _F32),
        pltpu.VMEM((NTW,), _F32),
        pltpu.VMEM((NTW, 8), _F32),
    ],
)
def _prep_kernel(degp, xt, zeros8, t1, dis, d0b, d1b, colsb, disb, t1b):
    base = _wid() * NTW
    pltpu.sync_copy(degp.at[0, pl.ds(base, NTW)], d0b)
    pltpu.sync_copy(degp.at[1, pl.ds(base, NTW)], d1b)
    for k in range(5):
        pltpu.sync_copy(xt.at[k, pl.ds(base, NTW)], colsb.at[k])
    pltpu.sync_copy(zeros8.at[pl.ds(base, NTW), :], t1b)
    io0 = _iota16()

    def body(i, _):
        sl = pl.ds(i * 16, 16)
        dv = d0b[sl] + d1b[sl] + 1.0
        y = _rsqrt16(dv)
        disb[sl] = y
        rowi = io0 + i * 16
        for k in range(5):
            plsc.store_scatter(t1b, [rowi, _full16(k)], _bf16r(colsb[k, sl]) * y)
        return 0
    lax.fori_loop(0, NTW // 16, body, 0)

    pltpu.sync_copy(disb, dis.at[pl.ds(base, NTW)])
    pltpu.sync_copy(t1b, t1.at[pl.ds(base, NTW), :])


def _make_agg(n_iters):
    @functools.partial(
        pl.kernel,
        out_type=jax.ShapeDtypeStruct((NC * NP, 8), _F32),
        mesh=_MESH,
        compiler_params=_PARAMS,
        scratch_types=[
            pltpu.VMEM((KS, NB), _I32),
            pltpu.VMEM((KS, NB), _I32),
            pltpu.VMEM((KS, NB, 8), _F32),
            pltpu.VMEM_SHARED((NP, 8), _F32),
            pltpu.VMEM_SHARED((NP, 8), _F32),
            pltpu.SemaphoreType.DMA,
            pltpu.SemaphoreType.DMA,
        ],
    )
    def agg_kernel(table, src2d, dst2d, zeros8, part,
                   sidx, didx, rows, acc, tbl, semg, sems):
        c = lax.axis_index("c")
        s = lax.axis_index("s")
        w = _wid()

        pltpu.sync_copy(zeros8.at[pl.ds(s * NPC, NPC), :],
                        acc.at[pl.ds(s * NPC, NPC), :])
        pltpu.sync_copy(table.at[pl.ds(s * NPC, NPC), :],
                        tbl.at[pl.ds(s * NPC, NPC), :])
        plsc.subcore_barrier()

        rows_per_worker = n_iters * KS

        def body(g, _):
            rbase = w * rows_per_worker + g * KS
            pltpu.sync_copy(src2d.at[pl.ds(rbase, KS)], sidx)
            pltpu.sync_copy(dst2d.at[pl.ds(rbase, KS)], didx)
            gd = [
                pltpu.async_copy(tbl.at[sidx.at[j]], rows.at[j], semg)
                for j in range(KS)
            ]
            sd = []
            for j in range(KS):
                gd[j].wait()
                sd.append(
                    pltpu.async_copy(rows.at[j], acc.at[didx.at[j]],
                                     sems, add=True))
            for d in sd:
                d.wait()
            return 0
        lax.fori_loop(0, n_iters, body, 0)

        plsc.subcore_barrier()
        pltpu.sync_copy(acc.at[pl.ds(s * NPC, NPC), :],
                        part.at[pl.ds(c * NP + s * NPC, NPC), :])

    return agg_kernel


@functools.partial(
    pl.kernel,
    out_type=jax.ShapeDtypeStruct((NP, 8), _F32),
    mesh=_MESH,
    compiler_params=_PARAMS,
    scratch_types=[
        pltpu.VMEM((NTW, 8), _F32),
        pltpu.VMEM((NTW, 8), _F32),
        pltpu.VMEM((NTW, 8), _F32),
        pltpu.VMEM((NTW,), _F32),
        pltpu.VMEM((NTW, 8), _F32),
        pltpu.VMEM((80,), _F32),
        pltpu.VMEM((16,), _F32),
        pltpu.VMEM((48,), _F32),
    ],
)
def _dense1_kernel(p1p, t1, dis, w1, b1, w2, zeros8, t2,
                   p0b, p1b, t1b, disb, t2b, w1b, b1b, w2b):
    base = _wid() * NTW
    pltpu.sync_copy(p1p.at[pl.ds(base, NTW), :], p0b)
    pltpu.sync_copy(p1p.at[pl.ds(NP + base, NTW), :], p1b)
    pltpu.sync_copy(t1.at[pl.ds(base, NTW), :], t1b)
    pltpu.sync_copy(dis.at[pl.ds(base, NTW)], disb)
    pltpu.sync_copy(w1, w1b)
    pltpu.sync_copy(b1, b1b)
    pltpu.sync_copy(w2, w2b)
    pltpu.sync_copy(zeros8.at[pl.ds(base, NTW), :], t2b)
    for t in range(5):
        w1b[pl.ds(t * 16, 16)] = _bf16r(w1b[pl.ds(t * 16, 16)])
    for t in range(3):
        w2b[pl.ds(t * 16, 16)] = _bf16r(w2b[pl.ds(t * 16, 16)])
    io0 = _iota16()
    zero16 = jnp.zeros((16,), dtype=_F32)

    def body(i, _):
        sl = pl.ds(i * 16, 16)
        y = disb[sl]
        rowi = io0 + i * 16
        u = []
        for cidx in range(5):
            fc = _full16(cidx)
            col = (plsc.load_gather(p0b, [rowi, fc])
                   + plsc.load_gather(p1b, [rowi, fc])
                   + plsc.load_gather(t1b, [rowi, fc]))
            u.append(y * col)

        def col_body(j, accs):
            a0, a1, a2 = accs
            jv = jnp.full((16,), j, dtype=_I32)
            o = plsc.load_gather(b1b, [jv])
            for cidx in range(5):
                o = o + u[cidx] * plsc.load_gather(w1b, [jv + cidx * 16])
            r = _bf16r(jnp.maximum(o, 0.0))
            j3 = j * 3
            a0 = a0 + r * plsc.load_gather(w2b, [jnp.full((16,), j3, _I32)])
            a1 = a1 + r * plsc.load_gather(w2b, [jnp.full((16,), j3 + 1, _I32)])
            a2 = a2 + r * plsc.load_gather(w2b, [jnp.full((16,), j3 + 2, _I32)])
            return (a0, a1, a2)
        a0, a1, a2 = lax.fori_loop(0, 16, col_body, (zero16, zero16, zero16))
        plsc.store_scatter(t2b, [rowi, _full16(0)], a0 * y)
        plsc.store_scatter(t2b, [rowi, _full16(1)], a1 * y)
        plsc.store_scatter(t2b, [rowi, _full16(2)], a2 * y)
        return 0
    lax.fori_loop(0, NTW // 16, body, 0)

    pltpu.sync_copy(t2b, t2.at[pl.ds(base, NTW), :])


@functools.partial(
    pl.kernel,
    out_type=jax.ShapeDtypeStruct((3, NP), _F32),
    mesh=_MESH,
    compiler_params=_PARAMS,
    scratch_types=[
        pltpu.VMEM((NTW, 8), _F32),
        pltpu.VMEM((NTW, 8), _F32),
        pltpu.VMEM((NTW, 8), _F32),
        pltpu.VMEM((NTW,), _F32),
        pltpu.VMEM((4, NTW), _F32),
        pltpu.VMEM((16,), _F32),
    ],
)
def _dense2_kernel(p2p, t2, dis, b2, outc,
                   p0b, p1b, t2b, disb, ocb, b2b):
    base = _wid() * NTW
    pltpu.sync_copy(p2p.at[pl.ds(base, NTW), :], p0b)
    pltpu.sync_copy(p2p.at[pl.ds(NP + base, NTW), :], p1b)
    pltpu.sync_copy(t2.at[pl.ds(base, NTW), :], t2b)
    pltpu.sync_copy(dis.at[pl.ds(base, NTW)], disb)
    pltpu.sync_copy(b2, b2b)
    io0 = _iota16()

    def body(i, _):
        sl = pl.ds(i * 16, 16)
        y = disb[sl]
        rowi = io0 + i * 16
        for k in range(3):
            fc = _full16(k)
            col = (plsc.load_gather(p0b, [rowi, fc])
                   + plsc.load_gather(p1b, [rowi, fc])
                   + plsc.load_gather(t2b, [rowi, fc]))
            ocb[k, sl] = y * col + _bcast(b2b, k)
        return 0
    lax.fori_loop(0, NTW // 16, body, 0)

    for k in range(3):
        pltpu.sync_copy(ocb.at[k], outc.at[k, pl.ds(base, NTW)])


def kernel(x, edge_index, W1, b1, W2, b2):
    n_edges = edge_index.shape[1]
    epw = NB * KS
    n_iters = -(-n_edges // (NW * epw))
    ep = n_iters * NW * epw

    npad = ep - n_edges
    padi = (N_NODES + jnp.arange(npad, dtype=_I32) % (NP - N_NODES))
    pad = jnp.stack([padi, padi])
    eip = jnp.concatenate([edge_index.astype(_I32), pad], axis=1)
    src2d = eip[0].reshape(ep // NB, NB)
    dst2d = eip[1].reshape(ep // NB, NB)

    xt = jnp.pad(x, ((0, NP - x.shape[0]), (0, 0))).T
    z1 = jnp.zeros((NP,), _F32)
    z8 = jnp.zeros((NP, 8), _F32)

    agg = _make_agg(n_iters)
    degp = _make_deg(n_iters)(dst2d, z1)
    t1, dis = _prep_kernel(degp, xt, z8)
    p1p = agg(t1, src2d, dst2d, z8)
    t2 = _dense1_kernel(p1p, t1, dis, W1.reshape(-1), b1, W2.reshape(-1), z8)
    p2p = agg(t2, src2d, dst2d, z8)
    outc = _dense2_kernel(p2p, t2, dis, jnp.pad(b2, (0, 13)))
    return outc[:, :N_NODES].T


# speedup vs baseline: 172.3924x; 1.0692x over previous
import functools

import jax
import jax.numpy as jnp
from jax import lax
from jax.experimental import pallas as pl
from jax.experimental.pallas import tpu as pltpu
from jax.experimental.pallas import tpu_sc as plsc

N_NODES = 100000
NC, NS = 2, 16
NW = NC * NS
NB = 128
KS = 16
KSD = 32
NTW = 3136
NP = NTW * NW
NPC = NP // NS

_MESH = plsc.VectorSubcoreMesh(
    core_axis_name="c", subcore_axis_name="s", num_cores=NC, num_subcores=NS)
_PARAMS = pltpu.CompilerParams(use_tc_tiling_on_sc=False,
                               needs_layout_passes=False)

_F32 = jnp.float32
_I32 = jnp.int32


def _wid():
    return lax.axis_index("c") * NS + lax.axis_index("s")


def _iota16():
    return lax.broadcasted_iota(_I32, (16,), 0)


def _full16(v):
    return jnp.full((16,), v, dtype=_I32)


def _rsqrt16(v):
    i = plsc.bitcast(v, _I32)
    i = jnp.int32(0x5F3759DF) - lax.shift_right_arithmetic(i, 1)
    y = plsc.bitcast(i, _F32)
    for _ in range(3):
        y = y * (1.5 - 0.5 * v * y * y)
    return y


def _bf16r(v):
    u = plsc.bitcast(v, _I32)
    r = u + jnp.int32(0x7FFF) + (lax.shift_right_logical(u, 16) & 1)
    r = r & jnp.int32(-65536)
    return plsc.bitcast(r, _F32)


def _bcast(ref, *idx):
    return plsc.load_gather(ref, [_full16(i) for i in idx])


def _make_deg(n_iters):
    @functools.partial(
        pl.kernel,
        out_type=jax.ShapeDtypeStruct((NC, NP), _F32),
        mesh=_MESH,
        compiler_params=_PARAMS,
        scratch_types=[
            pltpu.VMEM((KSD, NB), _I32),
            pltpu.VMEM((NP,), _F32),
            pltpu.VMEM((NP // 128, ), _F32),
            pltpu.VMEM((NP // 128, ), _F32),
            pltpu.VMEM_SHARED((NS, NP // 8), _F32),
        ],
    )
    def deg_kernel(dst2d, zeros1, degp, idxb, hist, abuf, bbuf, spm):
        c = lax.axis_index("c")
        s = lax.axis_index("s")
        w = _wid()
        ones16 = jnp.full((16,), 1.0, dtype=_F32)

        pltpu.sync_copy(zeros1, hist)
        rows_per_worker = n_iters * KSD

        def body(g, _):
            rbase = w * rows_per_worker + g * KSD
            pltpu.sync_copy(dst2d.at[pl.ds(rbase, KSD)], idxb)
            for j in range(KSD):
                def inner(k, _):
                    iv = idxb[j, pl.ds(k * 16, 16)]
                    plsc.addupdate_scatter(hist, [iv], ones16)
                    return 0
                lax.fori_loop(0, NB // 16, inner, 0)
            return 0
        lax.fori_loop(0, n_iters, body, 0)

        ch = NP // 8
        nred = ch // NS
        for r in range(8):
            pltpu.sync_copy(hist.at[pl.ds(r * ch, ch)], spm.at[s])
            plsc.subcore_barrier()
            pltpu.sync_copy(spm.at[0, pl.ds(s * nred, nred)], abuf)
            for slot in range(1, NS):
                pltpu.sync_copy(spm.at[slot, pl.ds(s * nred, nred)], bbuf)

                def acc(i, _):
                    sl = pl.ds(i * 16, 16)
                    abuf[sl] = abuf[sl] + bbuf[sl]
                    return 0
                lax.fori_loop(0, nred // 16, acc, 0)
            pltpu.sync_copy(abuf, degp.at[c, pl.ds(r * ch + s * nred, nred)])
            plsc.subcore_barrier()

    return deg_kernel


@functools.partial(
    pl.kernel,
    out_type=(jax.ShapeDtypeStruct((NP, 8), _F32),
              jax.ShapeDtypeStruct((NP,), _F32)),
    mesh=_MESH,
    compiler_params=_PARAMS,
    scratch_types=[
        pltpu.VMEM((NTW,), _F32),
        pltpu.VMEM((NTW,), _F32),
        pltpu.VMEM((5, NTW), _F32),
        pltpu.VMEM((NTW,), _F32),
        pltpu.VMEM((NTW, 8), _F32),
    ],
)
def _prep_kernel(degp, xt, zeros8, t1, dis, d0b, d1b, colsb, disb, t1b):
    base = _wid() * NTW
    pltpu.sync_copy(degp.at[0, pl.ds(base, NTW)], d0b)
    pltpu.sync_copy(degp.at[1, pl.ds(base, NTW)], d1b)
    for k in range(5):
        pltpu.sync_copy(xt.at[k, pl.ds(base, NTW)], colsb.at[k])
    pltpu.sync_copy(zeros8.at[pl.ds(base, NTW), :], t1b)
    io0 = _iota16()

    def body(i, _):
        sl = pl.ds(i * 16, 16)
        dv = d0b[sl] + d1b[sl] + 1.0
        y = _rsqrt16(dv)
        disb[sl] = y
        rowi = io0 + i * 16
        for k in range(5):
            plsc.store_scatter(t1b, [rowi, _full16(k)], _bf16r(colsb[k, sl]) * y)
        return 0
    lax.fori_loop(0, NTW // 16, body, 0)

    pltpu.sync_copy(disb, dis.at[pl.ds(base, NTW)])
    pltpu.sync_copy(t1b, t1.at[pl.ds(base, NTW), :])


def _make_agg(n_iters):
    @functools.partial(
        pl.kernel,
        out_type=jax.ShapeDtypeStruct((NC * NP, 8), _F32),
        mesh=_MESH,
        compiler_params=_PARAMS,
        scratch_types=[
            pltpu.VMEM((2 * KS, NB), _I32),
            pltpu.VMEM((KS, NB, 8), _F32),
            pltpu.VMEM_SHARED((NP, 8), _F32),
            pltpu.VMEM_SHARED((NP, 8), _F32),
            pltpu.SemaphoreType.DMA,
            pltpu.SemaphoreType.DMA,
        ],
    )
    def agg_kernel(table, e2d, zeros8, part,
                   eidx, rows, acc, tbl, semg, sems):
        c = lax.axis_index("c")
        s = lax.axis_index("s")
        w = _wid()

        pltpu.sync_copy(zeros8.at[pl.ds(s * NPC, NPC), :],
                        acc.at[pl.ds(s * NPC, NPC), :])
        pltpu.sync_copy(table.at[pl.ds(s * NPC, NPC), :],
                        tbl.at[pl.ds(s * NPC, NPC), :])
        plsc.subcore_barrier()

        rows_per_worker = n_iters * KS

        def body(g, _):
            rbase = 2 * (w * rows_per_worker + g * KS)
            pltpu.sync_copy(e2d.at[pl.ds(rbase, 2 * KS)], eidx)
            gd = [
                pltpu.async_copy(tbl.at[eidx.at[2 * j]], rows.at[j], semg)
                for j in range(KS)
            ]
            sd = []
            for j in range(KS):
                gd[j].wait()
                sd.append(
                    pltpu.async_copy(rows.at[j], acc.at[eidx.at[2 * j + 1]],
                                     sems, add=True))
            for d in sd:
                d.wait()
            return 0
        lax.fori_loop(0, n_iters, body, 0)

        plsc.subcore_barrier()
        pltpu.sync_copy(acc.at[pl.ds(s * NPC, NPC), :],
                        part.at[pl.ds(c * NP + s * NPC, NPC), :])

    return agg_kernel


@functools.partial(
    pl.kernel,
    out_type=jax.ShapeDtypeStruct((NP, 8), _F32),
    mesh=_MESH,
    compiler_params=_PARAMS,
    scratch_types=[
        pltpu.VMEM((NTW, 8), _F32),
        pltpu.VMEM((NTW, 8), _F32),
        pltpu.VMEM((NTW, 8), _F32),
        pltpu.VMEM((NTW,), _F32),
        pltpu.VMEM((NTW, 8), _F32),
        pltpu.VMEM((80,), _F32),
        pltpu.VMEM((16,), _F32),
        pltpu.VMEM((48,), _F32),
    ],
)
def _dense1_kernel(p1p, t1, dis, w1, b1, w2, zeros8, t2,
                   p0b, p1b, t1b, disb, t2b, w1b, b1b, w2b):
    base = _wid() * NTW
    pltpu.sync_copy(p1p.at[pl.ds(base, NTW), :], p0b)
    pltpu.sync_copy(p1p.at[pl.ds(NP + base, NTW), :], p1b)
    pltpu.sync_copy(t1.at[pl.ds(base, NTW), :], t1b)
    pltpu.sync_copy(dis.at[pl.ds(base, NTW)], disb)
    pltpu.sync_copy(w1, w1b)
    pltpu.sync_copy(b1, b1b)
    pltpu.sync_copy(w2, w2b)
    pltpu.sync_copy(zeros8.at[pl.ds(base, NTW), :], t2b)
    for t in range(5):
        w1b[pl.ds(t * 16, 16)] = _bf16r(w1b[pl.ds(t * 16, 16)])
    for t in range(3):
        w2b[pl.ds(t * 16, 16)] = _bf16r(w2b[pl.ds(t * 16, 16)])
    io0 = _iota16()
    zero16 = jnp.zeros((16,), dtype=_F32)

    def body(i, _):
        sl = pl.ds(i * 16, 16)
        y = disb[sl]
        rowi = io0 + i * 16
        u = []
        for cidx in range(5):
            fc = _full16(cidx)
            col = (plsc.load_gather(p0b, [rowi, fc])
                   + plsc.load_gather(p1b, [rowi, fc])
                   + plsc.load_gather(t1b, [rowi, fc]))
            u.append(y * col)

        def col_body(j, accs):
            a0, a1, a2 = accs
            jv = jnp.full((16,), j, dtype=_I32)
            o = plsc.load_gather(b1b, [jv])
            for cidx in range(5):
                o = o + u[cidx] * plsc.load_gather(w1b, [jv + cidx * 16])
            r = _bf16r(jnp.maximum(o, 0.0))
            j3 = j * 3
            a0 = a0 + r * plsc.load_gather(w2b, [jnp.full((16,), j3, _I32)])
            a1 = a1 + r * plsc.load_gather(w2b, [jnp.full((16,), j3 + 1, _I32)])
            a2 = a2 + r * plsc.load_gather(w2b, [jnp.full((16,), j3 + 2, _I32)])
            return (a0, a1, a2)
        a0, a1, a2 = lax.fori_loop(0, 16, col_body, (zero16, zero16, zero16))
        plsc.store_scatter(t2b, [rowi, _full16(0)], a0 * y)
        plsc.store_scatter(t2b, [rowi, _full16(1)], a1 * y)
        plsc.store_scatter(t2b, [rowi, _full16(2)], a2 * y)
        return 0
    lax.fori_loop(0, NTW // 16, body, 0)

    pltpu.sync_copy(t2b, t2.at[pl.ds(base, NTW), :])


@functools.partial(
    pl.kernel,
    out_type=jax.ShapeDtypeStruct((3, NP), _F32),
    mesh=_MESH,
    compiler_params=_PARAMS,
    scratch_types=[
        pltpu.VMEM((NTW, 8), _F32),
        pltpu.VMEM((NTW, 8), _F32),
        pltpu.VMEM((NTW, 8), _F32),
        pltpu.VMEM((NTW,), _F32),
        pltpu.VMEM((4, NTW), _F32),
        pltpu.VMEM((16,), _F32),
    ],
)
def _dense2_kernel(p2p, t2, dis, b2, outc,
                   p0b, p1b, t2b, disb, ocb, b2b):
    base = _wid() * NTW
    pltpu.sync_copy(p2p.at[pl.ds(base, NTW), :], p0b)
    pltpu.sync_copy(p2p.at[pl.ds(NP + base, NTW), :], p1b)
    pltpu.sync_copy(t2.at[pl.ds(base, NTW), :], t2b)
    pltpu.sync_copy(dis.at[pl.ds(base, NTW)], disb)
    pltpu.sync_copy(b2, b2b)
    io0 = _iota16()

    def body(i, _):
        sl = pl.ds(i * 16, 16)
        y = disb[sl]
        rowi = io0 + i * 16
        for k in range(3):
            fc = _full16(k)
            col = (plsc.load_gather(p0b, [rowi, fc])
                   + plsc.load_gather(p1b, [rowi, fc])
                   + plsc.load_gather(t2b, [rowi, fc]))
            ocb[k, sl] = y * col + _bcast(b2b, k)
        return 0
    lax.fori_loop(0, NTW // 16, body, 0)

    for k in range(3):
        pltpu.sync_copy(ocb.at[k], outc.at[k, pl.ds(base, NTW)])


def kernel(x, edge_index, W1, b1, W2, b2):
    n_edges = edge_index.shape[1]
    epwd = NB * KSD
    n_iters_d = -(-n_edges // (NW * epwd))
    ep = n_iters_d * NW * epwd
    n_iters = ep // (NW * NB * KS)

    npad = ep - n_edges
    padi = (N_NODES + jnp.arange(npad, dtype=_I32) % (NP - N_NODES))
    pad = jnp.stack([padi, padi])
    eip = jnp.concatenate([edge_index.astype(_I32), pad], axis=1)
    dst2d = eip[1].reshape(ep // NB, NB)
    e2d = jnp.stack([eip[0].reshape(ep // NB, NB), dst2d],
                    axis=1).reshape(2 * ep // NB, NB)

    xt = jnp.pad(x, ((0, NP - x.shape[0]), (0, 0))).T
    z1 = jnp.zeros((NP,), _F32)
    z8 = jnp.zeros((NP, 8), _F32)

    agg = _make_agg(n_iters)
    degp = _make_deg(n_iters_d)(dst2d, z1)
    t1, dis = _prep_kernel(degp, xt, z8)
    p1p = agg(t1, e2d, z8)
    t2 = _dense1_kernel(p1p, t1, dis, W1.reshape(-1), b1, W2.reshape(-1), z8)
    p2p = agg(t2, e2d, z8)
    outc = _dense2_kernel(p2p, t2, dis, jnp.pad(b2, (0, 13)))
    return outc[:, :N_NODES].T
